# baseline (speedup 1.0000x reference)
import jax
import jax.numpy as jnp
from jax import lax
from jax.experimental import pallas as pl
from jax.experimental.pallas import tpu as pltpu
from jax.experimental.pallas import tpu_sc as plsc

NUM_EMBEDDINGS = 1000000
EMBED_DIM = 64
EPS = 1e-5

NC = 2
NS = 16
NW = NC * NS
L = 16

B = 4096 * 200
PER_W = B // NW
CHUNK = 512
N_CHUNKS = PER_W // CHUNK
IDX_ROWS = CHUNK // 128
GROUPS = CHUNK // L


def _rsqrt(x):
    i = plsc.bitcast(x, jnp.int32)
    i = jnp.int32(0x5F3759DF) - lax.shift_right_logical(i, 1)
    y = plsc.bitcast(i, jnp.float32)
    for _ in range(3):
        y = y * (1.5 - 0.5 * (x * y * y))
    return y


def _body(x_hbm, table_hbm, gamma_hbm, beta_hbm, out_hbm,
          idx_v, rows_v, gamma_v, beta_v, sem):
    wid = lax.axis_index("s") * NC + lax.axis_index("c")
    pltpu.sync_copy(gamma_hbm, gamma_v)
    pltpu.sync_copy(beta_hbm, beta_v)
    idx_row0 = wid * (PER_W // 128)
    out_row0 = wid * PER_W

    gvecs = [gamma_v[pl.ds(k * L, L)] for k in range(EMBED_DIM // L)]
    bvecs = [beta_v[pl.ds(k * L, L)] for k in range(EMBED_DIM // L)]

    def chunk_body(ci, carry):
        pltpu.sync_copy(x_hbm.at[pl.ds(idx_row0 + ci * IDX_ROWS, IDX_ROWS)],
                        idx_v)
        copies = [
            pltpu.async_copy(table_hbm.at[idx_v.at[j]],
                             rows_v.at[pl.ds(j * 128, 128)], sem)
            for j in range(IDX_ROWS)
        ]
        for cp in copies:
            cp.wait()

        def group_body(g, gcarry):
            ridx = g * L + lax.iota(jnp.int32, L)
            s = jnp.zeros((L,), jnp.float32)
            q = jnp.zeros((L,), jnp.float32)
            for j in range(EMBED_DIM):
                cj = plsc.load_gather(
                    rows_v, [ridx, jnp.full((L,), j, jnp.int32)])
                s = s + cj
                q = q + cj * cj
            mean = s * (1.0 / EMBED_DIM)
            var = q * (1.0 / EMBED_DIM) - mean * mean
            rstd = _rsqrt(var + EPS)
            for j in range(EMBED_DIM):
                col = jnp.full((L,), j, jnp.int32)
                cj = plsc.load_gather(rows_v, [ridx, col])
                gj = gvecs[j // L][j % L]
                bj = bvecs[j // L][j % L]
                o = (cj - mean) * (rstd * gj) + bj
                plsc.store_scatter(rows_v, [ridx, col], o)
            return gcarry

        lax.fori_loop(0, GROUPS, group_body, 0)
        pltpu.sync_copy(rows_v,
                        out_hbm.at[pl.ds(out_row0 + ci * CHUNK, CHUNK)])
        return carry

    lax.fori_loop(0, N_CHUNKS, chunk_body, 0)


@jax.jit
def kernel(x, table, gamma, beta):
    xf = x.reshape(B // 128, 128)
    mesh = plsc.VectorSubcoreMesh(core_axis_name="c", subcore_axis_name="s",
                                  num_cores=NC, num_subcores=NS)
    out = pl.kernel(
        _body,
        out_type=jax.ShapeDtypeStruct((B, EMBED_DIM), jnp.float32),
        mesh=mesh,
        compiler_params=pltpu.CompilerParams(needs_layout_passes=False,
                                             use_tc_tiling_on_sc=False),
        scratch_types=[
            pltpu.VMEM((IDX_ROWS, 128), jnp.int32),
            pltpu.VMEM((CHUNK, EMBED_DIM), jnp.float32),
            pltpu.VMEM((EMBED_DIM,), jnp.float32),
            pltpu.VMEM((EMBED_DIM,), jnp.float32),
            pltpu.SemaphoreType.DMA,
        ],
    )(xf, table, gamma, beta)
    return out.reshape(x.shape[0], x.shape[1], EMBED_DIM)


# speedup vs baseline: 1.6543x; 1.6543x over previous
import jax
import jax.numpy as jnp
from jax import lax
from jax.experimental import pallas as pl
from jax.experimental.pallas import tpu as pltpu
from jax.experimental.pallas import tpu_sc as plsc

NUM_EMBEDDINGS = 1000000
EMBED_DIM = 64
EPS = 1e-5

NC = 2
NS = 16
NW = NC * NS
L = 16

B = 4096 * 200
PER_W = B // NW
CHUNK = 512
N_CHUNKS = PER_W // CHUNK
IDX_ROWS = CHUNK // 128
GROUPS = CHUNK // L


def _rsqrt(x):
    i = plsc.bitcast(x, jnp.int32)
    i = jnp.int32(0x5F3759DF) - lax.shift_right_logical(i, 1)
    y = plsc.bitcast(i, jnp.float32)
    for _ in range(3):
        y = y * (1.5 - 0.5 * (x * y * y))
    return y


def _body(x_hbm, table_hbm, gamma_hbm, beta_hbm, out_hbm,
          idx_v, rows_v, gamma_v, beta_v, sem):
    wid = lax.axis_index("s") * NC + lax.axis_index("c")
    pltpu.sync_copy(gamma_hbm, gamma_v)
    pltpu.sync_copy(beta_hbm, beta_v)
    idx_row0 = wid * (PER_W // 128)
    out_row0 = wid * PER_W


    def chunk_body(ci, carry):
        pltpu.sync_copy(x_hbm.at[pl.ds(idx_row0 + ci * IDX_ROWS, IDX_ROWS)],
                        idx_v)
        copies = [
            pltpu.async_copy(table_hbm.at[idx_v.at[j]],
                             rows_v.at[pl.ds(j * 128, 128)], sem)
            for j in range(IDX_ROWS)
        ]
        for cp in copies:
            cp.wait()

        def group_body(g, gcarry):
            ridx = g * L + lax.iota(jnp.int32, L)
            lane = lax.iota(jnp.int32, L)
            cols = [(lane + j) & (EMBED_DIM - 1) for j in range(EMBED_DIM)]
            acc = [jnp.zeros((L,), jnp.float32) for _ in range(8)]
            for j in range(EMBED_DIM):
                cj = plsc.load_gather(rows_v, [ridx, cols[j]])
                acc[2 * (j % 4)] = acc[2 * (j % 4)] + cj
                acc[2 * (j % 4) + 1] = acc[2 * (j % 4) + 1] + cj * cj
            s = (acc[0] + acc[2]) + (acc[4] + acc[6])
            q = (acc[1] + acc[3]) + (acc[5] + acc[7])
            mean = s * (1.0 / EMBED_DIM)
            var = q * (1.0 / EMBED_DIM) - mean * mean
            rstd = _rsqrt(var + EPS)
            for j in range(EMBED_DIM):
                cj = plsc.load_gather(rows_v, [ridx, cols[j]])
                gj = plsc.load_gather(gamma_v, [cols[j]])
                bj = plsc.load_gather(beta_v, [cols[j]])
                o = (cj - mean) * (rstd * gj) + bj
                plsc.store_scatter(rows_v, [ridx, cols[j]], o)
            return gcarry

        lax.fori_loop(0, GROUPS, group_body, 0)
        pltpu.sync_copy(rows_v,
                        out_hbm.at[pl.ds(out_row0 + ci * CHUNK, CHUNK)])
        return carry

    lax.fori_loop(0, N_CHUNKS, chunk_body, 0)


@jax.jit
def kernel(x, table, gamma, beta):
    xf = x.reshape(B // 128, 128)
    mesh = plsc.VectorSubcoreMesh(core_axis_name="c", subcore_axis_name="s",
                                  num_cores=NC, num_subcores=NS)
    out = pl.kernel(
        _body,
        out_type=jax.ShapeDtypeStruct((B, EMBED_DIM), jnp.float32),
        mesh=mesh,
        compiler_params=pltpu.CompilerParams(needs_layout_passes=False,
                                             use_tc_tiling_on_sc=False),
        scratch_types=[
            pltpu.VMEM((IDX_ROWS, 128), jnp.int32),
            pltpu.VMEM((CHUNK, EMBED_DIM), jnp.float32),
            pltpu.VMEM((EMBED_DIM,), jnp.float32),
            pltpu.VMEM((EMBED_DIM,), jnp.float32),
            pltpu.SemaphoreType.DMA,
        ],
    )(xf, table, gamma, beta)
    return out.reshape(x.shape[0], x.shape[1], EMBED_DIM)


# speedup vs baseline: 2.4707x; 1.4935x over previous
import functools

import jax
import jax.numpy as jnp
from jax import lax
from jax.experimental import pallas as pl
from jax.experimental.pallas import tpu as pltpu
from jax.experimental.pallas import tpu_sc as plsc

NUM_EMBEDDINGS = 1000000
EMBED_DIM = 64
EPS = 1e-5

NC = 2
NS = 16
NW = NC * NS

B = 4096 * 200
PER_W = B // NW
CHUNK = 512
N_CHUNKS = PER_W // CHUNK
IDX_ROWS = CHUNK // 128

BLK_B = 32
RB = BLK_B * 200 * EMBED_DIM // 128


def _gather_body(x_hbm, table_hbm, out_hbm, idx_v, rows_v, gsem0, gsem1):
    wid = lax.axis_index("s") * NC + lax.axis_index("c")
    idx_row0 = wid * (PER_W // 128)
    out_row0 = wid * PER_W
    gsems = (gsem0, gsem1)

    def load_idx(ci, b):
        pltpu.sync_copy(
            x_hbm.at[pl.ds(idx_row0 + ci * IDX_ROWS, IDX_ROWS)], idx_v.at[b])

    def fire(b):
        for j in range(IDX_ROWS):
            pltpu.async_copy(table_hbm.at[idx_v.at[b, j]],
                             rows_v.at[b, pl.ds(j * 128, 128)], gsems[b])

    def wait_gathers(b):
        for j in range(IDX_ROWS):
            pltpu.make_async_copy(table_hbm.at[idx_v.at[b, j]],
                                  rows_v.at[b, pl.ds(j * 128, 128)],
                                  gsems[b]).wait()

    def copy_out(ci, b):
        pltpu.sync_copy(rows_v.at[b],
                        out_hbm.at[pl.ds(out_row0 + ci * CHUNK, CHUNK)])

    def step(ci, b):
        nb = 1 - b
        load_idx(ci + 1, nb)
        fire(nb)
        wait_gathers(b)
        copy_out(ci, b)

    load_idx(0, 0)
    fire(0)

    def pair_body(k, carry):
        step(2 * k, 0)
        step(2 * k + 1, 1)
        return carry

    lax.fori_loop(0, N_CHUNKS // 2 - 1, pair_body, 0)
    step(N_CHUNKS - 2, 0)
    wait_gathers(1)
    copy_out(N_CHUNKS - 1, 1)


def _sc_gather(xf, table):
    mesh = plsc.VectorSubcoreMesh(core_axis_name="c", subcore_axis_name="s",
                                  num_cores=NC, num_subcores=NS)
    return pl.kernel(
        _gather_body,
        out_type=jax.ShapeDtypeStruct((B, EMBED_DIM), jnp.float32),
        mesh=mesh,
        compiler_params=pltpu.CompilerParams(needs_layout_passes=False,
                                             use_tc_tiling_on_sc=False),
        scratch_types=[
            pltpu.VMEM((2, IDX_ROWS, 128), jnp.int32),
            pltpu.VMEM((2, CHUNK, EMBED_DIM), jnp.float32),
            pltpu.SemaphoreType.DMA,
            pltpu.SemaphoreType.DMA,
        ],
    )(xf, table)


def _ln_body(g2_ref, gamma_ref, beta_ref, out_ref):
    x = g2_ref[...]
    g = gamma_ref[0, :]
    b = beta_ref[0, :]
    mean = jnp.mean(x, axis=1, keepdims=True)
    xc = x - mean
    var = jnp.mean(xc * xc, axis=1, keepdims=True)
    o = xc * lax.rsqrt(var + EPS) * g + b
    out_ref[...] = o.reshape(BLK_B, 200, EMBED_DIM)


def _tc_layernorm(g2, gamma2, beta2):
    return pl.pallas_call(
        _ln_body,
        grid=(4096 // BLK_B,),
        in_specs=[
            pl.BlockSpec((BLK_B * 200, EMBED_DIM), lambda i: (i, 0)),
            pl.BlockSpec((1, EMBED_DIM), lambda i: (0, 0)),
            pl.BlockSpec((1, EMBED_DIM), lambda i: (0, 0)),
        ],
        out_specs=pl.BlockSpec((BLK_B, 200, EMBED_DIM), lambda i: (i, 0, 0)),
        out_shape=jax.ShapeDtypeStruct((4096, 200, EMBED_DIM), jnp.float32),
    )(g2, gamma2, beta2)


@jax.jit
def kernel(x, table, gamma, beta):
    xf = x.astype(jnp.int32).reshape(B // 128, 128)
    g2 = _sc_gather(xf, table)
    return _tc_layernorm(g2, gamma.reshape(1, EMBED_DIM),
                         beta.reshape(1, EMBED_DIM))
